# Initial kernel scaffold; baseline (speedup 1.0000x reference)
import jax
import jax.numpy as jnp
from jax.experimental import pallas as pl


def kernel(in_feat, W, b, edge_index):
    raise NotImplementedError("write your pallas kernel here")


# speedup vs baseline: 3.2020x; 3.2020x over previous
import functools
import math

import jax
import jax.numpy as jnp
import numpy as np
from jax import lax
from jax.experimental import pallas as pl
from jax.experimental.pallas import tpu as pltpu
from jax.experimental.pallas import tpu_sc as plsc

N = 10000
E = 160000
IN_DIM = 128
H = 64
D_SCALES = 2

NC = 2
NS = 16
NW = NC * NS

NPAD = 10240
STRIPE = NPAD // NS
DUMP = 10200
CHUNK = 128
KCH = 40
EPAD = NW * KCH * CHUNK


def _beta_wavelet_thetas(d):
    thetas = []
    eval_max = 2.0
    for i in range(d):
        p = np.array([1.0])
        for _ in range(i):
            p = np.polymul(p, np.array([0.5, 0.0]))
        for _ in range(d - i):
            p = np.polymul(p, np.array([-0.5, 1.0]))
        beta = math.gamma(i + 1) * math.gamma(d + 1 - i) / math.gamma(d + 2)
        p = p / (eval_max * beta)
        asc = p[::-1]
        thetas.append([float(asc[j]) for j in range(d)])
    return thetas


_TH = _beta_wavelet_thetas(D_SCALES)
A_U, B_U = _TH[0][0] + _TH[0][1], -_TH[0][1]
A_V, B_V = _TH[1][0] + _TH[1][1], -_TH[1][1]

_SC_MESH = plsc.VectorSubcoreMesh(
    core_axis_name="c", subcore_axis_name="s", num_cores=NC, num_subcores=NS
)
_SC_PARAMS = pltpu.CompilerParams(use_tc_tiling_on_sc=False)


@functools.partial(
    pl.kernel,
    out_type=jax.ShapeDtypeStruct((NC, NPAD, 16), jnp.float32),
    mesh=_SC_MESH,
    scratch_types=[
        pltpu.VMEM((KCH, CHUNK), jnp.int32),
        pltpu.VMEM((CHUNK, 16), jnp.float32),
        pltpu.VMEM_SHARED((NPAD, 16), jnp.float32),
        pltpu.SemaphoreType.DMA,
    ],
    compiler_params=_SC_PARAMS,
)
def _deg_kernel(src_hbm, ones_hbm, zeros_hbm, out_hbm, idx_v, ones_v, deg_sh, sem):
    cid = lax.axis_index("c")
    sid = lax.axis_index("s")
    wid = cid * NS + sid
    pltpu.sync_copy(zeros_hbm, deg_sh.at[pl.ds(sid * STRIPE, STRIPE)])
    pltpu.sync_copy(src_hbm.at[wid], idx_v)
    pltpu.sync_copy(ones_hbm, ones_v)
    plsc.subcore_barrier()

    def body(j, _):
        pltpu.sync_copy(ones_v, deg_sh.at[idx_v.at[j]], add=True)
        return 0

    lax.fori_loop(0, KCH, body, 0)
    plsc.subcore_barrier()
    pltpu.sync_copy(
        deg_sh.at[pl.ds(sid * STRIPE, STRIPE)],
        out_hbm.at[cid, pl.ds(sid * STRIPE, STRIPE)],
    )


@functools.partial(
    pl.kernel,
    out_type=jax.ShapeDtypeStruct((NC, NPAD, H), jnp.float32),
    mesh=_SC_MESH,
    scratch_types=[
        pltpu.VMEM((KCH, CHUNK), jnp.int32),
        pltpu.VMEM((KCH, CHUNK), jnp.int32),
        pltpu.VMEM((CHUNK, H), jnp.float32),
        pltpu.VMEM_SHARED((NPAD, H), jnp.float32),
        pltpu.SemaphoreType.DMA,
    ],
    compiler_params=_SC_PARAMS,
)
def _agg_kernel(hs_hbm, src_hbm, dst_hbm, zeros_hbm, out_hbm,
                src_v, dst_v, rows_v, agg_sh, sem):
    cid = lax.axis_index("c")
    sid = lax.axis_index("s")
    wid = cid * NS + sid
    pltpu.sync_copy(zeros_hbm, agg_sh.at[pl.ds(sid * STRIPE, STRIPE)])
    pltpu.sync_copy(src_hbm.at[wid], src_v)
    pltpu.sync_copy(dst_hbm.at[wid], dst_v)
    plsc.subcore_barrier()

    def body(j, _):
        pltpu.async_copy(hs_hbm.at[src_v.at[j]], rows_v, sem).wait()
        pltpu.sync_copy(rows_v, agg_sh.at[dst_v.at[j]], add=True)
        return 0

    lax.fori_loop(0, KCH, body, 0)
    plsc.subcore_barrier()
    pltpu.sync_copy(
        agg_sh.at[pl.ds(sid * STRIPE, STRIPE)],
        out_hbm.at[cid, pl.ds(sid * STRIPE, STRIPE)],
    )


def _h_body(x_ref, w_ref, b_ref, d0_ref, d1_ref, h_ref, hs_ref):
    h = jnp.dot(x_ref[...], w_ref[...], preferred_element_type=jnp.float32)
    h = h + b_ref[...]
    h = jnp.where(h >= 0, h, 0.01 * h)
    deg = 1.0 + d0_ref[...] + d1_ref[...]
    dinv = lax.rsqrt(jnp.maximum(deg, 1.0))
    h_ref[...] = h
    hs_ref[...] = h * dinv


def _combine_body(h_ref, hs_ref, s0_ref, s1_ref, d0_ref, d1_ref, u_ref, v_ref):
    deg = 1.0 + d0_ref[...] + d1_ref[...]
    dinv = lax.rsqrt(jnp.maximum(deg, 1.0))
    g = dinv * (s0_ref[...] + s1_ref[...] + hs_ref[...])
    h = h_ref[...]
    u_ref[...] = A_U * h + B_U * g
    v_ref[...] = A_V * h + B_V * g


def _recon_body(ui_ref, uj_ref, vi_ref, vj_ref, o0_ref, o1_ref):
    dn = (((1,), (1,)), ((), ()))
    o0_ref[...] = lax.dot_general(ui_ref[...], uj_ref[...], dn,
                                  preferred_element_type=jnp.float32)
    o1_ref[...] = lax.dot_general(vi_ref[...], vj_ref[...], dn,
                                  preferred_element_type=jnp.float32)


def kernel(in_feat, W, b, edge_index):
    src = edge_index[0]
    dst = edge_index[1]
    pad = jnp.full((EPAD - E,), DUMP, dtype=src.dtype)
    src_p = jnp.concatenate([src, pad]).reshape(NW, KCH, CHUNK)
    dst_p = jnp.concatenate([dst, pad]).reshape(NW, KCH, CHUNK)

    ones16 = jnp.ones((CHUNK, 16), jnp.float32)
    zeros16 = jnp.zeros((STRIPE, 16), jnp.float32)
    zeros64 = jnp.zeros((STRIPE, H), jnp.float32)

    deg_parts = _deg_kernel(src_p, ones16, zeros16)
    d0 = deg_parts[0, :, 0:1]
    d1 = deg_parts[1, :, 0:1]

    x_pad = jnp.pad(in_feat, ((0, NPAD - N), (0, 0)))
    bm = 1024
    h_pad, hs_pad = pl.pallas_call(
        _h_body,
        grid=(NPAD // bm,),
        in_specs=[
            pl.BlockSpec((bm, IN_DIM), lambda i: (i, 0)),
            pl.BlockSpec((IN_DIM, H), lambda i: (0, 0)),
            pl.BlockSpec((1, H), lambda i: (0, 0)),
            pl.BlockSpec((bm, 1), lambda i: (i, 0)),
            pl.BlockSpec((bm, 1), lambda i: (i, 0)),
        ],
        out_specs=[
            pl.BlockSpec((bm, H), lambda i: (i, 0)),
            pl.BlockSpec((bm, H), lambda i: (i, 0)),
        ],
        out_shape=[
            jax.ShapeDtypeStruct((NPAD, H), jnp.float32),
            jax.ShapeDtypeStruct((NPAD, H), jnp.float32),
        ],
    )(x_pad, W, b.reshape(1, H), d0, d1)

    agg_parts = _agg_kernel(hs_pad, src_p, dst_p, zeros64)

    bc = 1000
    u, v = pl.pallas_call(
        _combine_body,
        grid=(N // bc,),
        in_specs=[
            pl.BlockSpec((bc, H), lambda i: (i, 0)),
            pl.BlockSpec((bc, H), lambda i: (i, 0)),
            pl.BlockSpec((bc, H), lambda i: (i, 0)),
            pl.BlockSpec((bc, H), lambda i: (i, 0)),
            pl.BlockSpec((bc, 1), lambda i: (i, 0)),
            pl.BlockSpec((bc, 1), lambda i: (i, 0)),
        ],
        out_specs=[
            pl.BlockSpec((bc, H), lambda i: (i, 0)),
            pl.BlockSpec((bc, H), lambda i: (i, 0)),
        ],
        out_shape=[
            jax.ShapeDtypeStruct((N, H), jnp.float32),
            jax.ShapeDtypeStruct((N, H), jnp.float32),
        ],
    )(h_pad[:N], hs_pad[:N], agg_parts[0, :N], agg_parts[1, :N],
      d0[:N], d1[:N])

    bo = 1024
    o0, o1 = pl.pallas_call(
        _recon_body,
        grid=(pl.cdiv(N, bo), pl.cdiv(N, bo)),
        in_specs=[
            pl.BlockSpec((bo, H), lambda i, j: (i, 0)),
            pl.BlockSpec((bo, H), lambda i, j: (j, 0)),
            pl.BlockSpec((bo, H), lambda i, j: (i, 0)),
            pl.BlockSpec((bo, H), lambda i, j: (j, 0)),
        ],
        out_specs=[
            pl.BlockSpec((bo, bo), lambda i, j: (i, j)),
            pl.BlockSpec((bo, bo), lambda i, j: (i, j)),
        ],
        out_shape=[
            jax.ShapeDtypeStruct((N, N), jnp.float32),
            jax.ShapeDtypeStruct((N, N), jnp.float32),
        ],
    )(u, u, v, v)
    return (o0, o1)


# speedup vs baseline: 3.5294x; 1.1022x over previous
import functools
import math

import jax
import jax.numpy as jnp
import numpy as np
from jax import lax
from jax.experimental import pallas as pl
from jax.experimental.pallas import tpu as pltpu
from jax.experimental.pallas import tpu_sc as plsc

N = 10000
E = 160000
IN_DIM = 128
H = 64
D_SCALES = 2

NC = 2
NS = 16
NW = NC * NS

NPAD = 10240
STRIPE = NPAD // NS
DUMP = 10200
CHUNK = 128
KCH = 40
EPAD = NW * KCH * CHUNK


def _beta_wavelet_thetas(d):
    thetas = []
    eval_max = 2.0
    for i in range(d):
        p = np.array([1.0])
        for _ in range(i):
            p = np.polymul(p, np.array([0.5, 0.0]))
        for _ in range(d - i):
            p = np.polymul(p, np.array([-0.5, 1.0]))
        beta = math.gamma(i + 1) * math.gamma(d + 1 - i) / math.gamma(d + 2)
        p = p / (eval_max * beta)
        asc = p[::-1]
        thetas.append([float(asc[j]) for j in range(d)])
    return thetas


_TH = _beta_wavelet_thetas(D_SCALES)
A_U, B_U = _TH[0][0] + _TH[0][1], -_TH[0][1]
A_V, B_V = _TH[1][0] + _TH[1][1], -_TH[1][1]

_SC_MESH = plsc.VectorSubcoreMesh(
    core_axis_name="c", subcore_axis_name="s", num_cores=NC, num_subcores=NS
)
_SC_PARAMS = pltpu.CompilerParams(use_tc_tiling_on_sc=False)


@functools.partial(
    pl.kernel,
    out_type=jax.ShapeDtypeStruct((NC, NPAD, 16), jnp.float32),
    mesh=_SC_MESH,
    scratch_types=[
        pltpu.VMEM((KCH, CHUNK), jnp.int32),
        pltpu.VMEM((CHUNK, 16), jnp.float32),
        pltpu.VMEM_SHARED((NPAD, 16), jnp.float32),
        pltpu.SemaphoreType.DMA,
    ],
    compiler_params=_SC_PARAMS,
)
def _deg_kernel(src_hbm, ones_hbm, zeros_hbm, out_hbm, idx_v, ones_v, deg_sh, sem):
    cid = lax.axis_index("c")
    sid = lax.axis_index("s")
    wid = cid * NS + sid
    pltpu.sync_copy(zeros_hbm, deg_sh.at[pl.ds(sid * STRIPE, STRIPE)])
    pltpu.sync_copy(src_hbm.at[wid], idx_v)
    pltpu.sync_copy(ones_hbm, ones_v)
    plsc.subcore_barrier()

    def body(j, _):
        pltpu.sync_copy(ones_v, deg_sh.at[idx_v.at[j]], add=True)
        return 0

    lax.fori_loop(0, KCH, body, 0)
    plsc.subcore_barrier()
    pltpu.sync_copy(
        deg_sh.at[pl.ds(sid * STRIPE, STRIPE)],
        out_hbm.at[cid, pl.ds(sid * STRIPE, STRIPE)],
    )


@functools.partial(
    pl.kernel,
    out_type=jax.ShapeDtypeStruct((NC, NPAD, H), jnp.float32),
    mesh=_SC_MESH,
    scratch_types=[
        pltpu.VMEM((KCH, CHUNK), jnp.int32),
        pltpu.VMEM((KCH, CHUNK), jnp.int32),
        pltpu.VMEM((CHUNK, H), jnp.float32),
        pltpu.VMEM((CHUNK, H), jnp.float32),
        pltpu.VMEM_SHARED((NPAD, H), jnp.float32),
        pltpu.SemaphoreType.DMA,
        pltpu.SemaphoreType.DMA,
    ],
    compiler_params=_SC_PARAMS,
)
def _agg_kernel(hs_hbm, src_hbm, dst_hbm, zeros_hbm, out_hbm,
                src_v, dst_v, rows0, rows1, agg_sh, sem0, sem1):
    cid = lax.axis_index("c")
    sid = lax.axis_index("s")
    wid = cid * NS + sid
    pltpu.sync_copy(zeros_hbm, agg_sh.at[pl.ds(sid * STRIPE, STRIPE)])
    pltpu.sync_copy(src_hbm.at[wid], src_v)
    pltpu.sync_copy(dst_hbm.at[wid], dst_v)
    plsc.subcore_barrier()
    pltpu.async_copy(hs_hbm.at[src_v.at[0]], rows0, sem0)

    def body(jj, _):
        j0 = 2 * jj
        j1 = j0 + 1
        pltpu.make_async_copy(hs_hbm.at[src_v.at[j0]], rows0, sem0).wait()
        pltpu.async_copy(hs_hbm.at[src_v.at[j1]], rows1, sem1)
        pltpu.sync_copy(rows0, agg_sh.at[dst_v.at[j0]], add=True)

        @pl.when(jj < KCH // 2 - 1)
        def _prefetch():
            pltpu.async_copy(hs_hbm.at[src_v.at[j0 + 2]], rows0, sem0)

        pltpu.make_async_copy(hs_hbm.at[src_v.at[j1]], rows1, sem1).wait()
        pltpu.sync_copy(rows1, agg_sh.at[dst_v.at[j1]], add=True)
        return 0

    lax.fori_loop(0, KCH // 2, body, 0)
    plsc.subcore_barrier()
    pltpu.sync_copy(
        agg_sh.at[pl.ds(sid * STRIPE, STRIPE)],
        out_hbm.at[cid, pl.ds(sid * STRIPE, STRIPE)],
    )


def _h_body(x_ref, w_ref, b_ref, h_ref):
    h = jnp.dot(x_ref[...], w_ref[...], preferred_element_type=jnp.float32)
    h = h + b_ref[...]
    h_ref[...] = jnp.where(h >= 0, h, 0.01 * h)


def _hs_body(h_ref, d0_ref, d1_ref, hs_ref):
    deg = 1.0 + d0_ref[...] + d1_ref[...]
    dinv = lax.rsqrt(jnp.maximum(deg, 1.0))
    hs_ref[...] = h_ref[...] * dinv


def _combine_body(h_ref, hs_ref, s0_ref, s1_ref, d0_ref, d1_ref, u_ref, v_ref):
    deg = 1.0 + d0_ref[...] + d1_ref[...]
    dinv = lax.rsqrt(jnp.maximum(deg, 1.0))
    g = dinv * (s0_ref[...] + s1_ref[...] + hs_ref[...])
    h = h_ref[...]
    u_ref[...] = A_U * h + B_U * g
    v_ref[...] = A_V * h + B_V * g


def _recon_body(ui_ref, uj_ref, vi_ref, vj_ref, o0_ref, o1_ref):
    dn = (((1,), (1,)), ((), ()))
    o0_ref[...] = lax.dot_general(ui_ref[...], uj_ref[...], dn,
                                  preferred_element_type=jnp.float32)
    o1_ref[...] = lax.dot_general(vi_ref[...], vj_ref[...], dn,
                                  preferred_element_type=jnp.float32)


def kernel(in_feat, W, b, edge_index):
    src = edge_index[0]
    dst = edge_index[1]
    npd = EPAD - E
    pad_dump = (N + jnp.arange(npd, dtype=src.dtype) % (NPAD - N))
    pad_zero = jnp.zeros((npd,), dtype=src.dtype)
    src_deg = jnp.concatenate([src, pad_dump]).reshape(NW, KCH, CHUNK)
    src_gat = jnp.concatenate([src, pad_zero]).reshape(NW, KCH, CHUNK)
    dst_p = jnp.concatenate([dst, pad_dump]).reshape(NW, KCH, CHUNK)

    ones16 = jnp.ones((CHUNK, 16), jnp.float32)
    zeros16 = jnp.zeros((STRIPE, 16), jnp.float32)
    zeros64 = jnp.zeros((STRIPE, H), jnp.float32)

    deg_parts = _deg_kernel(src_deg, ones16, zeros16)
    d0 = deg_parts[0, :N, 0:1]
    d1 = deg_parts[1, :N, 0:1]

    bm = 1000
    h = pl.pallas_call(
        _h_body,
        grid=(N // bm,),
        in_specs=[
            pl.BlockSpec((bm, IN_DIM), lambda i: (i, 0)),
            pl.BlockSpec((IN_DIM, H), lambda i: (0, 0)),
            pl.BlockSpec((1, H), lambda i: (0, 0)),
        ],
        out_specs=pl.BlockSpec((bm, H), lambda i: (i, 0)),
        out_shape=jax.ShapeDtypeStruct((N, H), jnp.float32),
    )(in_feat, W, b.reshape(1, H))

    hs = pl.pallas_call(
        _hs_body,
        grid=(N // bm,),
        in_specs=[
            pl.BlockSpec((bm, H), lambda i: (i, 0)),
            pl.BlockSpec((bm, 1), lambda i: (i, 0)),
            pl.BlockSpec((bm, 1), lambda i: (i, 0)),
        ],
        out_specs=pl.BlockSpec((bm, H), lambda i: (i, 0)),
        out_shape=jax.ShapeDtypeStruct((N, H), jnp.float32),
    )(h, d0, d1)

    agg_parts = _agg_kernel(hs, src_gat, dst_p, zeros64)

    bc = 1000
    u, v = pl.pallas_call(
        _combine_body,
        grid=(N // bc,),
        in_specs=[
            pl.BlockSpec((bc, H), lambda i: (i, 0)),
            pl.BlockSpec((bc, H), lambda i: (i, 0)),
            pl.BlockSpec((bc, H), lambda i: (i, 0)),
            pl.BlockSpec((bc, H), lambda i: (i, 0)),
            pl.BlockSpec((bc, 1), lambda i: (i, 0)),
            pl.BlockSpec((bc, 1), lambda i: (i, 0)),
        ],
        out_specs=[
            pl.BlockSpec((bc, H), lambda i: (i, 0)),
            pl.BlockSpec((bc, H), lambda i: (i, 0)),
        ],
        out_shape=[
            jax.ShapeDtypeStruct((N, H), jnp.float32),
            jax.ShapeDtypeStruct((N, H), jnp.float32),
        ],
    )(h, hs, agg_parts[0, :N], agg_parts[1, :N], d0, d1)

    bo = 1024
    o0, o1 = pl.pallas_call(
        _recon_body,
        grid=(pl.cdiv(N, bo), pl.cdiv(N, bo)),
        in_specs=[
            pl.BlockSpec((bo, H), lambda i, j: (i, 0)),
            pl.BlockSpec((bo, H), lambda i, j: (j, 0)),
            pl.BlockSpec((bo, H), lambda i, j: (i, 0)),
            pl.BlockSpec((bo, H), lambda i, j: (j, 0)),
        ],
        out_specs=[
            pl.BlockSpec((bo, bo), lambda i, j: (i, j)),
            pl.BlockSpec((bo, bo), lambda i, j: (i, j)),
        ],
        out_shape=[
            jax.ShapeDtypeStruct((N, N), jnp.float32),
            jax.ShapeDtypeStruct((N, N), jnp.float32),
        ],
    )(u, u, v, v)
    return (o0, o1)
